# baseline (speedup 1.0000x reference)
import functools

import jax
import jax.numpy as jnp
from jax import lax
from jax.experimental import pallas as pl
from jax.experimental.pallas import tpu as pltpu
from jax.experimental.pallas import tpu_sc as plsc

DIM = 128
NC = 2
NS = 16
NW = NC * NS
NBUF = 5
LAG = 2


@functools.partial(jax.jit, static_argnames=("n_pos", "n_sent"))
def _sc_gather(table, idx_t, n_pos, n_sent):
    cols = n_sent // NW
    mesh = plsc.VectorSubcoreMesh(core_axis_name="c", subcore_axis_name="s")

    @functools.partial(
        pl.kernel,
        mesh=mesh,
        out_type=jax.ShapeDtypeStruct((n_pos, n_sent, DIM), jnp.float32),
        scratch_types=[
            pltpu.VMEM((n_pos, cols), jnp.int32),
            pltpu.VMEM((NBUF, cols, DIM), jnp.float32),
            pltpu.SemaphoreType.DMA((NBUF,)),
            pltpu.SemaphoreType.DMA((NBUF,)),
        ],
    )
    def k(table_hbm, idx_hbm, out_hbm, idx_v, rows_v, gsems, osems):
        wid = lax.axis_index("s") * NC + lax.axis_index("c")
        col0 = wid * cols
        pltpu.sync_copy(idx_hbm.at[:, pl.ds(col0, cols)], idx_v)
        for b in range(NBUF):
            pltpu.make_async_copy(
                table_hbm.at[idx_v.at[b]], rows_v.at[b], gsems.at[b]
            ).start()

        def step(s, carry):
            for b in range(NBUF):
                j = s * NBUF + b
                pltpu.make_async_copy(
                    table_hbm.at[idx_v.at[j]], rows_v.at[b], gsems.at[b]
                ).wait()
                pltpu.make_async_copy(
                    rows_v.at[b], out_hbm.at[j, pl.ds(col0, cols)], osems.at[b]
                ).start()
                br = (b - LAG) % NBUF
                c = j + NBUF - LAG

                @pl.when(jnp.logical_and(c >= NBUF, c < n_pos))
                def _():
                    pltpu.make_async_copy(
                        rows_v.at[br], out_hbm.at[c - NBUF, pl.ds(col0, cols)],
                        osems.at[br],
                    ).wait()
                    pltpu.make_async_copy(
                        table_hbm.at[idx_v.at[c]], rows_v.at[br], gsems.at[br]
                    ).start()

            return carry

        lax.fori_loop(0, n_pos // NBUF, step, 0)
        for c in range(n_pos - NBUF, n_pos):
            pltpu.make_async_copy(
                rows_v.at[c % NBUF],
                out_hbm.at[c, pl.ds(col0, cols)],
                osems.at[c % NBUF],
            ).wait()

    return k(table, idx_t)


def kernel(word_vector, table):
    n_sent, n_pos = word_vector.shape
    idx_t = word_vector.T.astype(jnp.int32)
    out_t = _sc_gather(table, idx_t, n_pos, n_sent)
    return out_t.transpose(1, 0, 2)
